# baseline (speedup 1.0000x reference)
import functools

import jax
import jax.numpy as jnp
from jax import lax
from jax.experimental import pallas as pl
from jax.experimental.pallas import tpu as pltpu
from jax.experimental.pallas import tpu_sc as plsc

N = 10000
NP = 10240
E = 320000
EP = 327680
CH = 64
EROWS = EP // CH
NC = 2
NS = 16
NW = NC * NS
RPW = EROWS // NW
RPS = EROWS // NS
NROWS16 = 640
NPT = NP // NS
HW = 64


def _rsqrt_newton(v):
    i = plsc.bitcast(v, jnp.int32)
    i = jnp.int32(0x5F3759DF) - (i >> 1)
    y = plsc.bitcast(i, jnp.float32)
    for _ in range(3):
        y = y * (1.5 - 0.5 * v * y * y)
    return y


def _fori(lo, hi, body):
    lax.fori_loop(lo, hi, lambda i, c: (body(i), 0)[1], 0)


def _sc_pre(src2, dst2, w2):
    mesh = plsc.VectorSubcoreMesh(core_axis_name="c", subcore_axis_name="s")

    @functools.partial(
        pl.kernel,
        out_type=(
            jax.ShapeDtypeStruct((EROWS, CH), jnp.float32),
            jax.ShapeDtypeStruct((NROWS16, 16), jnp.float32),
        ),
        mesh=mesh,
        scratch_types=[
            pltpu.VMEM((RPS, CH), jnp.int32),
            pltpu.VMEM((RPS, CH), jnp.float32),
            pltpu.VMEM((RPW, CH), jnp.int32),
            pltpu.VMEM((RPW, CH), jnp.float32),
            pltpu.VMEM((NROWS16, 16), jnp.float32),
            pltpu.VMEM((5, 128), jnp.int32),
            pltpu.VMEM((40, 16), jnp.float32),
            pltpu.VMEM_SHARED((NROWS16, 16), jnp.float32),
        ],
        compiler_params=pltpu.CompilerParams(
            use_tc_tiling_on_sc=False, needs_layout_passes=False),
    )
    def k(src_h, dst_h, w_h, wn_h, d2_h, ibuf, fbuf, ibuf2, obuf, deg2,
          ridx, initbuf, deg_sh):
        c = lax.axis_index("c")
        s = lax.axis_index("s")
        wid = s * NC + c
        iota = lax.iota(jnp.int32, 16)
        ones = jnp.ones((16,), jnp.float32)
        zeros = jnp.zeros((16,), jnp.float32)

        @plsc.parallel_loop(0, NROWS16, unroll=4)
        def zero_row(r):
            deg2[r] = zeros
        for p in range(5):
            for q in range(8):
                ridx[p, pl.ds(q * 16, 16)] = iota + (p * 128 + q * 16)

        def ones_row(r):
            initbuf[r] = ones
        _fori(0, 40, ones_row)
        pltpu.sync_copy(initbuf, deg_sh.at[pl.ds(s * 40, 40)])

        pltpu.sync_copy(dst_h.at[pl.ds(s * RPS, RPS)], ibuf)
        pltpu.sync_copy(w_h.at[pl.ds(s * RPS, RPS)], fbuf)

        def deg_row(r):
            for g in range(CH // 16):
                d16 = ibuf[r, pl.ds(g * 16, 16)]
                wv = fbuf[r, pl.ds(g * 16, 16)]
                plsc.addupdate_scatter(deg2, [d16 >> 4, d16 & 15], wv)
        _fori(0, RPS, deg_row)

        plsc.subcore_barrier()
        for p in range(5):
            pltpu.sync_copy(deg2.at[pl.ds(p * 128, 128)],
                            deg_sh.at[ridx.at[p]], add=True)
        plsc.subcore_barrier()

        pltpu.sync_copy(deg_sh, deg2)

        @plsc.parallel_loop(0, NROWS16, unroll=4)
        def dinv_row(r):
            deg2[r] = _rsqrt_newton(deg2[r])

        base = wid * RPW
        pltpu.sync_copy(src_h.at[pl.ds(base, RPW)], ibuf.at[pl.ds(0, RPW)])
        pltpu.sync_copy(dst_h.at[pl.ds(base, RPW)], ibuf2)
        pltpu.sync_copy(w_h.at[pl.ds(base, RPW)], fbuf.at[pl.ds(0, RPW)])

        @plsc.parallel_loop(0, RPW, unroll=2)
        def wn_row(r):
            for g in range(CH // 16):
                sl = pl.ds(g * 16, 16)
                s16 = ibuf[r, sl]
                d16 = ibuf2[r, sl]
                wv = fbuf[r, sl]
                da = plsc.load_gather(deg2, [s16 >> 4, s16 & 15])
                db = plsc.load_gather(deg2, [d16 >> 4, d16 & 15])
                obuf[r, sl] = da * wv * db
        pltpu.sync_copy(obuf, wn_h.at[pl.ds(base, RPW)])

        @pl.when(c == 0)
        def _():
            def d2_row(r):
                v = deg2[s * 40 + r]
                initbuf[r] = v * v
            _fori(0, 40, d2_row)
            pltpu.sync_copy(initbuf, d2_h.at[pl.ds(s * 40, 40)])

    return k(src2, dst2, w2)


def _sc_agg(h, src2, dst2, wn2, npass):
    mesh = plsc.VectorSubcoreMesh(core_axis_name="c", subcore_axis_name="s")
    kd = HW // 16
    nb = 4

    @functools.partial(
        pl.kernel,
        out_type=jax.ShapeDtypeStruct((NC, npass, NP, HW), jnp.float32),
        mesh=mesh,
        scratch_types=(
            [
                pltpu.VMEM((RPW, CH), jnp.int32),
                pltpu.VMEM((RPW, CH), jnp.int32),
                pltpu.VMEM((RPW, CH), jnp.float32),
            ]
            + [pltpu.VMEM((CH, HW), jnp.float32) for _ in range(nb)]
            + [
                pltpu.VMEM((16, HW), jnp.float32),
                pltpu.VMEM_SHARED((NP, HW), jnp.float32),
                pltpu.VMEM_SHARED((NP, HW), jnp.float32),
            ]
            + [pltpu.SemaphoreType.DMA for _ in range(2 * nb)]
        ),
        compiler_params=pltpu.CompilerParams(
            use_tc_tiling_on_sc=False, needs_layout_passes=False),
    )
    def k(h_h, src_h, dst_h, wn_h, gp_h, src_v, dst_v, wn_v, *bufs):
        rows = bufs[:nb]
        zbuf = bufs[nb]
        h_sp = bufs[nb + 1]
        acc_sh = bufs[nb + 2]
        gsem = bufs[nb + 3:nb + 3 + nb]
        ssem = bufs[nb + 3 + nb:]
        c = lax.axis_index("c")
        s = lax.axis_index("s")
        wid = s * NC + c
        zeros = jnp.zeros((16,), jnp.float32)

        base = wid * RPW
        pltpu.sync_copy(src_h.at[pl.ds(base, RPW)], src_v)
        pltpu.sync_copy(dst_h.at[pl.ds(base, RPW)], dst_v)
        pltpu.sync_copy(wn_h.at[pl.ds(base, RPW)], wn_v)

        def zero_row(r):
            for kk in range(kd):
                zbuf[r, pl.ds(kk * 16, 16)] = zeros
        _fori(0, 16, zero_row)

        def scale(rbuf, j):
            jb = jnp.broadcast_to(j, (16,))

            @plsc.parallel_loop(0, CH // 16, unroll=4)
            def sgroup(g):
                g16 = g * 16
                for e in range(16):
                    col = jnp.broadcast_to(g16 + e, (16,))
                    bw = plsc.load_gather(wn_v, [jb, col])
                    for kk in range(kd):
                        sl = pl.ds(kk * 16, 16)
                        rbuf[g16 + e, sl] = rbuf[g16 + e, sl] * bw

        for p in range(npass):
            psl = pl.ds(s * NPT, NPT)
            pltpu.sync_copy(h_h.at[p].at[psl], h_sp.at[psl])

            def zero_acc(r):
                pltpu.sync_copy(zbuf, acc_sh.at[pl.ds(s * NPT + r * 16, 16)])
            _fori(0, NPT // 16, zero_acc)
            plsc.subcore_barrier()

            for b in range(nb):
                pltpu.async_copy(h_sp.at[src_v.at[b]], rows[b], gsem[b])

            def step(jj):
                for b in range(nb):
                    cb = nb * jj + b
                    pltpu.make_async_copy(h_sp.at[src_v.at[cb]],
                                          rows[b], gsem[b]).wait()
                    scale(rows[b], cb)
                    pltpu.async_copy(rows[b], acc_sh.at[dst_v.at[cb]],
                                     ssem[b], add=True)

                    @pl.when(jj < RPW // nb - 1)
                    def _():
                        pltpu.make_async_copy(rows[b],
                                              acc_sh.at[dst_v.at[cb]],
                                              ssem[b]).wait()
                        pltpu.async_copy(h_sp.at[src_v.at[cb + nb]],
                                         rows[b], gsem[b])
            _fori(0, RPW // nb, step)
            for b in range(nb):
                pltpu.make_async_copy(rows[b],
                                      acc_sh.at[dst_v.at[RPW - nb + b]],
                                      ssem[b]).wait()
            plsc.subcore_barrier()

            for r in range(NPT // 128):
                sl = pl.ds(s * NPT + r * 128, 128)
                pltpu.sync_copy(acc_sh.at[sl], gp_h.at[c, p].at[sl])
            plsc.subcore_barrier()

    return k(h, src2, dst2, wn2)


def _tc_mm_split(x, w):
    din = x.shape[1]
    blk = 2048

    def body(x_ref, w_ref, o_ref):
        r = jnp.dot(x_ref[...], w_ref[...], preferred_element_type=jnp.float32)
        o_ref[0] = r[:, :HW]
        o_ref[1] = r[:, HW:]

    return pl.pallas_call(
        body,
        grid=(NP // blk,),
        in_specs=[
            pl.BlockSpec((blk, din), lambda i: (i, 0)),
            pl.BlockSpec((din, 2 * HW), lambda i: (0, 0)),
        ],
        out_specs=pl.BlockSpec((2, blk, HW), lambda i: (0, i, 0)),
        out_shape=jax.ShapeDtypeStruct((2, NP, HW), jnp.float32),
    )(x, w)


def _tc_comb(gp, h, dinv2, b_pre, w, b_post):
    npin = gp.shape[1]
    dout = w.shape[1]
    blk = 2048

    def body(g_ref, h_ref, d2_ref, bpre_ref, w_ref, bpost_ref, o_ref):
        parts = []
        d2 = d2_ref[...]
        for p in range(npin):
            t = (g_ref[0, p] + g_ref[1, p] + d2 * h_ref[p]
                 + bpre_ref[:, p * HW:(p + 1) * HW])
            parts.append(t)
        a = jnp.maximum(jnp.concatenate(parts, axis=-1), 0.0)
        o_ref[...] = (jnp.dot(a, w_ref[...],
                              preferred_element_type=jnp.float32)
                      + bpost_ref[...])

    return pl.pallas_call(
        body,
        grid=(NP // blk,),
        in_specs=[
            pl.BlockSpec((NC, npin, blk, HW), lambda i: (0, 0, i, 0)),
            pl.BlockSpec((npin, blk, HW), lambda i: (0, i, 0)),
            pl.BlockSpec((blk, 1), lambda i: (i, 0)),
            pl.BlockSpec((1, npin * HW), lambda i: (0, 0)),
            pl.BlockSpec((npin * HW, dout), lambda i: (0, 0)),
            pl.BlockSpec((1, dout), lambda i: (0, 0)),
        ],
        out_specs=pl.BlockSpec((blk, dout), lambda i: (i, 0)),
        out_shape=jax.ShapeDtypeStruct((NP, dout), jnp.float32),
    )(gp, h, dinv2, b_pre, w, b_post)


def kernel(x, edge_index, edge_weight, W1, b1, W2, b2, Wl, bl):
    pad = EP - E
    izero = jnp.zeros((pad,), edge_index.dtype)
    src2 = jnp.concatenate([edge_index[0], izero]).reshape(EROWS, CH)
    dst2 = jnp.concatenate([edge_index[1], izero]).reshape(EROWS, CH)
    w2 = jnp.concatenate(
        [edge_weight, jnp.zeros((pad,), edge_weight.dtype)]).reshape(EROWS, CH)

    wn2, d2 = _sc_pre(src2, dst2, w2)
    dinv2 = d2.reshape(NP, 1)

    xp = jnp.concatenate(
        [x, jnp.zeros((NP - N, x.shape[1]), x.dtype)], axis=0)
    h1 = _tc_mm_split(xp, W1)
    g1 = _sc_agg(h1, src2, dst2, wn2, 2)
    h2 = _tc_comb(g1, h1, dinv2, b1.reshape(1, -1), W2,
                  jnp.zeros((1, W2.shape[1]), jnp.float32))
    h2 = h2.reshape(1, NP, HW)
    g2 = _sc_agg(h2, src2, dst2, wn2, 1)
    out = _tc_comb(g2, h2, dinv2, b2.reshape(1, -1), Wl,
                   bl.reshape(1, -1))
    return out[:N]


# speedup vs baseline: 1.1249x; 1.1249x over previous
import functools

import jax
import jax.numpy as jnp
from jax import lax
from jax.experimental import pallas as pl
from jax.experimental.pallas import tpu as pltpu
from jax.experimental.pallas import tpu_sc as plsc

N = 10000
NP = 10240
E = 320000
EP = 327680
CH = 64
EROWS = EP // CH
NC = 2
NS = 16
NW = NC * NS
RPW = EROWS // NW
RPS = EROWS // NS
NROWS16 = 640
NPT = NP // NS
HW = 64


def _rsqrt_newton(v):
    i = plsc.bitcast(v, jnp.int32)
    i = jnp.int32(0x5F3759DF) - (i >> 1)
    y = plsc.bitcast(i, jnp.float32)
    for _ in range(3):
        y = y * (1.5 - 0.5 * v * y * y)
    return y


def _fori(lo, hi, body):
    lax.fori_loop(lo, hi, lambda i, c: (body(i), 0)[1], 0)


def _sc_pre(src2, dst2, w2):
    mesh = plsc.VectorSubcoreMesh(core_axis_name="c", subcore_axis_name="s")

    @functools.partial(
        pl.kernel,
        out_type=(
            jax.ShapeDtypeStruct((EROWS, CH), jnp.float32),
            jax.ShapeDtypeStruct((NROWS16, 16), jnp.float32),
        ),
        mesh=mesh,
        scratch_types=[
            pltpu.VMEM((RPS, CH), jnp.int32),
            pltpu.VMEM((RPS, CH), jnp.float32),
            pltpu.VMEM((RPW, CH), jnp.int32),
            pltpu.VMEM((RPW, CH), jnp.float32),
            pltpu.VMEM((NROWS16, 16), jnp.float32),
            pltpu.VMEM((5, 128), jnp.int32),
            pltpu.VMEM((40, 16), jnp.float32),
            pltpu.VMEM_SHARED((NROWS16, 16), jnp.float32),
        ],
        compiler_params=pltpu.CompilerParams(
            use_tc_tiling_on_sc=False, needs_layout_passes=False),
    )
    def k(src_h, dst_h, w_h, wn_h, d2_h, ibuf, fbuf, ibuf2, obuf, deg2,
          ridx, initbuf, deg_sh):
        c = lax.axis_index("c")
        s = lax.axis_index("s")
        wid = s * NC + c
        iota = lax.iota(jnp.int32, 16)
        ones = jnp.ones((16,), jnp.float32)
        zeros = jnp.zeros((16,), jnp.float32)

        @plsc.parallel_loop(0, NROWS16, unroll=4)
        def zero_row(r):
            deg2[r] = zeros
        for p in range(5):
            for q in range(8):
                ridx[p, pl.ds(q * 16, 16)] = iota + (p * 128 + q * 16)

        def ones_row(r):
            initbuf[r] = ones
        _fori(0, 40, ones_row)
        pltpu.sync_copy(initbuf, deg_sh.at[pl.ds(s * 40, 40)])

        pltpu.sync_copy(dst_h.at[pl.ds(s * RPS, RPS)], ibuf)
        pltpu.sync_copy(w_h.at[pl.ds(s * RPS, RPS)], fbuf)

        def deg_row(r):
            for g in range(CH // 16):
                d16 = ibuf[r, pl.ds(g * 16, 16)]
                wv = fbuf[r, pl.ds(g * 16, 16)]
                plsc.addupdate_scatter(deg2, [d16 >> 4, d16 & 15], wv)
        _fori(0, RPS, deg_row)

        plsc.subcore_barrier()
        for p in range(5):
            pltpu.sync_copy(deg2.at[pl.ds(p * 128, 128)],
                            deg_sh.at[ridx.at[p]], add=True)
        plsc.subcore_barrier()

        pltpu.sync_copy(deg_sh, deg2)

        @plsc.parallel_loop(0, NROWS16, unroll=4)
        def dinv_row(r):
            deg2[r] = _rsqrt_newton(deg2[r])

        base = wid * RPW
        pltpu.sync_copy(src_h.at[pl.ds(base, RPW)], ibuf.at[pl.ds(0, RPW)])
        pltpu.sync_copy(dst_h.at[pl.ds(base, RPW)], ibuf2)
        pltpu.sync_copy(w_h.at[pl.ds(base, RPW)], fbuf.at[pl.ds(0, RPW)])

        @plsc.parallel_loop(0, RPW, unroll=2)
        def wn_row(r):
            for g in range(CH // 16):
                sl = pl.ds(g * 16, 16)
                s16 = ibuf[r, sl]
                d16 = ibuf2[r, sl]
                wv = fbuf[r, sl]
                da = plsc.load_gather(deg2, [s16 >> 4, s16 & 15])
                db = plsc.load_gather(deg2, [d16 >> 4, d16 & 15])
                obuf[r, sl] = da * wv * db
        pltpu.sync_copy(obuf, wn_h.at[pl.ds(base, RPW)])

        @pl.when(c == 0)
        def _():
            def d2_row(r):
                v = deg2[s * 40 + r]
                initbuf[r] = v * v
            _fori(0, 40, d2_row)
            pltpu.sync_copy(initbuf, d2_h.at[pl.ds(s * 40, 40)])

    return k(src2, dst2, w2)


def _sc_agg(h, src2, dst2, wn2, npass):
    mesh = plsc.VectorSubcoreMesh(core_axis_name="c", subcore_axis_name="s")
    kd = HW // 16
    nb = 4

    @functools.partial(
        pl.kernel,
        out_type=jax.ShapeDtypeStruct((NC, npass, NP, HW), jnp.float32),
        mesh=mesh,
        scratch_types=(
            [
                pltpu.VMEM((RPW, CH), jnp.int32),
                pltpu.VMEM((RPW, CH), jnp.int32),
                pltpu.VMEM((RPW, CH), jnp.float32),
            ]
            + [pltpu.VMEM((CH, HW), jnp.float32) for _ in range(nb)]
            + [
                pltpu.VMEM((16, HW), jnp.float32),
                pltpu.VMEM_SHARED((NP, HW), jnp.float32),
                pltpu.VMEM_SHARED((NP, HW), jnp.float32),
            ]
            + [pltpu.SemaphoreType.DMA for _ in range(2 * nb)]
        ),
        compiler_params=pltpu.CompilerParams(
            use_tc_tiling_on_sc=False, needs_layout_passes=False),
    )
    def k(h_h, src_h, dst_h, wn_h, gp_h, src_v, dst_v, wn_v, *bufs):
        rows = bufs[:nb]
        zbuf = bufs[nb]
        h_sp = bufs[nb + 1]
        acc_sh = bufs[nb + 2]
        gsem = bufs[nb + 3:nb + 3 + nb]
        ssem = bufs[nb + 3 + nb:]
        c = lax.axis_index("c")
        s = lax.axis_index("s")
        wid = s * NC + c
        zeros = jnp.zeros((16,), jnp.float32)

        base = wid * RPW
        pltpu.sync_copy(src_h.at[pl.ds(base, RPW)], src_v)
        pltpu.sync_copy(dst_h.at[pl.ds(base, RPW)], dst_v)
        pltpu.sync_copy(wn_h.at[pl.ds(base, RPW)], wn_v)

        def zero_row(r):
            for kk in range(kd):
                zbuf[r, pl.ds(kk * 16, 16)] = zeros
        _fori(0, 16, zero_row)

        def scale(rbuf, j):
            jb = jnp.broadcast_to(j, (16,))

            @plsc.parallel_loop(0, CH // 16, unroll=2)
            def sgroup(g):
                g16 = g * 16
                for e in range(16):
                    col = jnp.broadcast_to(g16 + e, (16,))
                    bw = plsc.load_gather(wn_v, [jb, col])
                    for kk in range(kd):
                        sl = pl.ds(kk * 16, 16)
                        rbuf[g16 + e, sl] = rbuf[g16 + e, sl] * bw

        for p in range(npass):
            psl = pl.ds(s * NPT, NPT)
            pltpu.sync_copy(h_h.at[p].at[psl], h_sp.at[psl])

            def zero_acc(r):
                pltpu.sync_copy(zbuf, acc_sh.at[pl.ds(s * NPT + r * 16, 16)])
            _fori(0, NPT // 16, zero_acc)
            plsc.subcore_barrier()

            for b in range(nb):
                pltpu.async_copy(h_sp.at[src_v.at[b]], rows[b], gsem[b])

            def step(jj):
                for b in range(nb):
                    cb = nb * jj + b
                    pltpu.make_async_copy(h_sp.at[src_v.at[cb]],
                                          rows[b], gsem[b]).wait()
                    scale(rows[b], cb)
                    pltpu.async_copy(rows[b], acc_sh.at[dst_v.at[cb]],
                                     ssem[b], add=True)

                    @pl.when(jj < RPW // nb - 1)
                    def _():
                        pltpu.make_async_copy(rows[b],
                                              acc_sh.at[dst_v.at[cb]],
                                              ssem[b]).wait()
                        pltpu.async_copy(h_sp.at[src_v.at[cb + nb]],
                                         rows[b], gsem[b])
            _fori(0, RPW // nb, step)
            for b in range(nb):
                pltpu.make_async_copy(rows[b],
                                      acc_sh.at[dst_v.at[RPW - nb + b]],
                                      ssem[b]).wait()
            plsc.subcore_barrier()

            for r in range(NPT // 128):
                sl = pl.ds(s * NPT + r * 128, 128)
                pltpu.sync_copy(acc_sh.at[sl], gp_h.at[c, p].at[sl])
            plsc.subcore_barrier()

    return k(h, src2, dst2, wn2)


def _tc_mm_split(x, w):
    din = x.shape[1]
    blk = 2048

    def body(x_ref, w_ref, o_ref):
        r = jnp.dot(x_ref[...], w_ref[...], preferred_element_type=jnp.float32)
        o_ref[0] = r[:, :HW]
        o_ref[1] = r[:, HW:]

    return pl.pallas_call(
        body,
        grid=(NP // blk,),
        in_specs=[
            pl.BlockSpec((blk, din), lambda i: (i, 0)),
            pl.BlockSpec((din, 2 * HW), lambda i: (0, 0)),
        ],
        out_specs=pl.BlockSpec((2, blk, HW), lambda i: (0, i, 0)),
        out_shape=jax.ShapeDtypeStruct((2, NP, HW), jnp.float32),
    )(x, w)


def _tc_comb(gp, h, dinv2, b_pre, w, b_post):
    npin = gp.shape[1]
    dout = w.shape[1]
    blk = 2048

    def body(g_ref, h_ref, d2_ref, bpre_ref, w_ref, bpost_ref, o_ref):
        parts = []
        d2 = d2_ref[...]
        for p in range(npin):
            t = (g_ref[0, p] + g_ref[1, p] + d2 * h_ref[p]
                 + bpre_ref[:, p * HW:(p + 1) * HW])
            parts.append(t)
        a = jnp.maximum(jnp.concatenate(parts, axis=-1), 0.0)
        o_ref[...] = (jnp.dot(a, w_ref[...],
                              preferred_element_type=jnp.float32)
                      + bpost_ref[...])

    return pl.pallas_call(
        body,
        grid=(NP // blk,),
        in_specs=[
            pl.BlockSpec((NC, npin, blk, HW), lambda i: (0, 0, i, 0)),
            pl.BlockSpec((npin, blk, HW), lambda i: (0, i, 0)),
            pl.BlockSpec((blk, 1), lambda i: (i, 0)),
            pl.BlockSpec((1, npin * HW), lambda i: (0, 0)),
            pl.BlockSpec((npin * HW, dout), lambda i: (0, 0)),
            pl.BlockSpec((1, dout), lambda i: (0, 0)),
        ],
        out_specs=pl.BlockSpec((blk, dout), lambda i: (i, 0)),
        out_shape=jax.ShapeDtypeStruct((NP, dout), jnp.float32),
    )(gp, h, dinv2, b_pre, w, b_post)


def kernel(x, edge_index, edge_weight, W1, b1, W2, b2, Wl, bl):
    pad = EP - E
    izero = jnp.zeros((pad,), edge_index.dtype)
    src2 = jnp.concatenate([edge_index[0], izero]).reshape(EROWS, CH)
    dst2 = jnp.concatenate([edge_index[1], izero]).reshape(EROWS, CH)
    w2 = jnp.concatenate(
        [edge_weight, jnp.zeros((pad,), edge_weight.dtype)]).reshape(EROWS, CH)

    wn2, d2 = _sc_pre(src2, dst2, w2)
    dinv2 = d2.reshape(NP, 1)

    xp = jnp.concatenate(
        [x, jnp.zeros((NP - N, x.shape[1]), x.dtype)], axis=0)
    h1 = _tc_mm_split(xp, W1)
    g1 = _sc_agg(h1, src2, dst2, wn2, 2)
    h2 = _tc_comb(g1, h1, dinv2, b1.reshape(1, -1), W2,
                  jnp.zeros((1, W2.shape[1]), jnp.float32))
    h2 = h2.reshape(1, NP, HW)
    g2 = _sc_agg(h2, src2, dst2, wn2, 1)
    out = _tc_comb(g2, h2, dinv2, b2.reshape(1, -1), Wl,
                   bl.reshape(1, -1))
    return out[:N]


# speedup vs baseline: 1.3304x; 1.1827x over previous
import functools

import jax
import jax.numpy as jnp
import numpy as np
from jax import lax
from jax.experimental import pallas as pl
from jax.experimental.pallas import tpu as pltpu
from jax.experimental.pallas import tpu_sc as plsc

N = 10000
NP = 10240
E = 320000
EP = 327680
CH = 64
EROWS = EP // CH
NC = 2
NS = 16
NW = NC * NS
RPW = EROWS // NW
RPS = EROWS // NS
NROWS16 = 640
NPT = NP // NS
HW = 64


def _rsqrt_newton(v):
    i = plsc.bitcast(v, jnp.int32)
    i = jnp.int32(0x5F3759DF) - (i >> 1)
    y = plsc.bitcast(i, jnp.float32)
    for _ in range(3):
        y = y * (1.5 - 0.5 * v * y * y)
    return y


def _fori(lo, hi, body):
    lax.fori_loop(lo, hi, lambda i, c: (body(i), 0)[1], 0)


def _sc_pre(src2, dst2, w2):
    mesh = plsc.VectorSubcoreMesh(core_axis_name="c", subcore_axis_name="s")

    @functools.partial(
        pl.kernel,
        out_type=(
            jax.ShapeDtypeStruct((EROWS, CH), jnp.float32),
            jax.ShapeDtypeStruct((NROWS16, 16), jnp.float32),
        ),
        mesh=mesh,
        scratch_types=[
            pltpu.VMEM((RPS, CH), jnp.int32),
            pltpu.VMEM((RPS, CH), jnp.float32),
            pltpu.VMEM((RPW, CH), jnp.int32),
            pltpu.VMEM((RPW, CH), jnp.float32),
            pltpu.VMEM((NROWS16, 16), jnp.float32),
            pltpu.VMEM((5, 128), jnp.int32),
            pltpu.VMEM((40, 16), jnp.float32),
            pltpu.VMEM_SHARED((NROWS16, 16), jnp.float32),
        ],
        compiler_params=pltpu.CompilerParams(
            use_tc_tiling_on_sc=False, needs_layout_passes=False),
    )
    def k(src_h, dst_h, w_h, wn_h, d2_h, ibuf, fbuf, ibuf2, obuf, deg2,
          ridx, initbuf, deg_sh):
        c = lax.axis_index("c")
        s = lax.axis_index("s")
        wid = s * NC + c
        iota = lax.iota(jnp.int32, 16)
        ones = jnp.ones((16,), jnp.float32)
        zeros = jnp.zeros((16,), jnp.float32)

        @plsc.parallel_loop(0, NROWS16, unroll=4)
        def zero_row(r):
            deg2[r] = zeros
        for p in range(5):
            for q in range(8):
                ridx[p, pl.ds(q * 16, 16)] = iota + (p * 128 + q * 16)

        def ones_row(r):
            initbuf[r] = ones
        _fori(0, 40, ones_row)
        pltpu.sync_copy(initbuf, deg_sh.at[pl.ds(s * 40, 40)])

        pltpu.sync_copy(dst_h.at[pl.ds(s * RPS, RPS)], ibuf)
        pltpu.sync_copy(w_h.at[pl.ds(s * RPS, RPS)], fbuf)

        def deg_row(r):
            for g in range(CH // 16):
                d16 = ibuf[r, pl.ds(g * 16, 16)]
                wv = fbuf[r, pl.ds(g * 16, 16)]
                plsc.addupdate_scatter(deg2, [d16 >> 4, d16 & 15], wv)
        _fori(0, RPS, deg_row)

        plsc.subcore_barrier()
        for p in range(5):
            pltpu.sync_copy(deg2.at[pl.ds(p * 128, 128)],
                            deg_sh.at[ridx.at[p]], add=True)
        plsc.subcore_barrier()

        pltpu.sync_copy(deg_sh, deg2)

        @plsc.parallel_loop(0, NROWS16, unroll=4)
        def dinv_row(r):
            deg2[r] = _rsqrt_newton(deg2[r])

        base = wid * RPW
        pltpu.sync_copy(src_h.at[pl.ds(base, RPW)], ibuf.at[pl.ds(0, RPW)])
        pltpu.sync_copy(dst_h.at[pl.ds(base, RPW)], ibuf2)
        pltpu.sync_copy(w_h.at[pl.ds(base, RPW)], fbuf.at[pl.ds(0, RPW)])

        @plsc.parallel_loop(0, RPW, unroll=2)
        def wn_row(r):
            for g in range(CH // 16):
                sl = pl.ds(g * 16, 16)
                s16 = ibuf[r, sl]
                d16 = ibuf2[r, sl]
                wv = fbuf[r, sl]
                da = plsc.load_gather(deg2, [s16 >> 4, s16 & 15])
                db = plsc.load_gather(deg2, [d16 >> 4, d16 & 15])
                obuf[r, sl] = da * wv * db
        pltpu.sync_copy(obuf, wn_h.at[pl.ds(base, RPW)])

        @pl.when(c == 0)
        def _():
            def d2_row(r):
                v = deg2[s * 40 + r]
                initbuf[r] = v * v
            _fori(0, 40, d2_row)
            pltpu.sync_copy(initbuf, d2_h.at[pl.ds(s * 40, 40)])

    return k(src2, dst2, w2)


def _sc_agg(h, src2, dst2, wn2, npass):
    mesh = plsc.VectorSubcoreMesh(core_axis_name="c", subcore_axis_name="s")
    nb = 4

    @functools.partial(
        pl.kernel,
        out_type=jax.ShapeDtypeStruct((NC, npass, NP, HW), jnp.float32),
        mesh=mesh,
        scratch_types=(
            [
                pltpu.VMEM((RPW, CH), jnp.int32),
                pltpu.VMEM((RPW, CH), jnp.int32),
                pltpu.VMEM((RPW, CH), jnp.float32),
            ]
            + [pltpu.VMEM((CH, HW), jnp.bfloat16) for _ in range(nb)]
            + [pltpu.VMEM((CH, HW), jnp.float32) for _ in range(nb)]
            + [
                pltpu.VMEM((16, HW), jnp.float32),
                pltpu.VMEM_SHARED((NP, HW), jnp.bfloat16),
                pltpu.VMEM_SHARED((NP, HW), jnp.float32),
            ]
            + [pltpu.SemaphoreType.DMA for _ in range(2 * nb)]
        ),
        compiler_params=pltpu.CompilerParams(
            use_tc_tiling_on_sc=False, needs_layout_passes=False),
    )
    def k(h_h, src_h, dst_h, wn_h, gp_h, src_v, dst_v, wn_v, *bufs):
        rows = bufs[:nb]
        outs = bufs[nb:2 * nb]
        zbuf = bufs[2 * nb]
        h_sp = bufs[2 * nb + 1]
        acc_sh = bufs[2 * nb + 2]
        gsem = bufs[2 * nb + 3:2 * nb + 3 + nb]
        ssem = bufs[2 * nb + 3 + nb:]
        c = lax.axis_index("c")
        s = lax.axis_index("s")
        wid = s * NC + c
        zeros = jnp.zeros((16,), jnp.float32)

        base = wid * RPW
        pltpu.sync_copy(src_h.at[pl.ds(base, RPW)], src_v)
        pltpu.sync_copy(dst_h.at[pl.ds(base, RPW)], dst_v)
        pltpu.sync_copy(wn_h.at[pl.ds(base, RPW)], wn_v)

        def zero_row(r):
            for kk in range(HW // 16):
                zbuf[r, pl.ds(kk * 16, 16)] = zeros
        _fori(0, 16, zero_row)

        def scale(rbuf, obuf, j):
            jb = jnp.broadcast_to(j, (16,))

            @plsc.parallel_loop(0, CH // 16, unroll=2)
            def sgroup(g):
                g16 = g * 16
                for e in range(16):
                    col = jnp.broadcast_to(g16 + e, (16,))
                    bw = plsc.load_gather(wn_v, [jb, col])
                    for hh in range(HW // 32):
                        v32 = rbuf[g16 + e, pl.ds(hh * 32, 32)]
                        ev, od = plsc.unpack(
                            v32, format=plsc.PackFormat.INTERLEAVED)
                        obuf[g16 + e, pl.ds(hh * 32, 16)] = ev * bw
                        obuf[g16 + e, pl.ds(hh * 32 + 16, 16)] = od * bw

        for p in range(npass):
            psl = pl.ds(s * NPT, NPT)
            pltpu.sync_copy(h_h.at[p].at[psl], h_sp.at[psl])

            def zero_acc(r):
                pltpu.sync_copy(zbuf, acc_sh.at[pl.ds(s * NPT + r * 16, 16)])
            _fori(0, NPT // 16, zero_acc)
            plsc.subcore_barrier()

            for b in range(nb):
                pltpu.async_copy(h_sp.at[src_v.at[b]], rows[b], gsem[b])

            def step(jj):
                for b in range(nb):
                    cb = nb * jj + b
                    pltpu.make_async_copy(h_sp.at[src_v.at[cb]],
                                          rows[b], gsem[b]).wait()

                    @pl.when(jj > 0)
                    def _():
                        pltpu.make_async_copy(outs[b],
                                              acc_sh.at[dst_v.at[cb - nb]],
                                              ssem[b]).wait()
                    scale(rows[b], outs[b], cb)

                    @pl.when(jj < RPW // nb - 1)
                    def _():
                        pltpu.async_copy(h_sp.at[src_v.at[cb + nb]],
                                         rows[b], gsem[b])
                    pltpu.async_copy(outs[b], acc_sh.at[dst_v.at[cb]],
                                     ssem[b], add=True)
            _fori(0, RPW // nb, step)
            for b in range(nb):
                pltpu.make_async_copy(outs[b],
                                      acc_sh.at[dst_v.at[RPW - nb + b]],
                                      ssem[b]).wait()
            plsc.subcore_barrier()

            for r in range(NPT // 128):
                sl = pl.ds(s * NPT + r * 128, 128)
                pltpu.sync_copy(acc_sh.at[sl], gp_h.at[c, p].at[sl])
            plsc.subcore_barrier()

    return k(h, src2, dst2, wn2)


def _tc_mm_split(x, w, wp):
    din = x.shape[1]
    blk = 2048

    def body(x_ref, w_ref, wp_ref, t_ref, hp_ref):
        xv = x_ref[...]
        r = jnp.dot(xv, w_ref[...], preferred_element_type=jnp.float32)
        rp = jnp.dot(xv, wp_ref[...], preferred_element_type=jnp.float32)
        t_ref[0] = r[:, :HW].astype(jnp.bfloat16)
        t_ref[1] = r[:, HW:].astype(jnp.bfloat16)
        hp_ref[0] = rp[:, :HW]
        hp_ref[1] = rp[:, HW:]

    return pl.pallas_call(
        body,
        grid=(NP // blk,),
        in_specs=[
            pl.BlockSpec((blk, din), lambda i: (i, 0)),
            pl.BlockSpec((din, 2 * HW), lambda i: (0, 0)),
            pl.BlockSpec((din, 2 * HW), lambda i: (0, 0)),
        ],
        out_specs=[
            pl.BlockSpec((2, blk, HW), lambda i: (0, i, 0)),
            pl.BlockSpec((2, blk, HW), lambda i: (0, i, 0)),
        ],
        out_shape=[
            jax.ShapeDtypeStruct((2, NP, HW), jnp.bfloat16),
            jax.ShapeDtypeStruct((2, NP, HW), jnp.float32),
        ],
    )(x, w, wp)


def _tc_comb1(gp, hperm, dinv2, b_pre, w, wp):
    npin = gp.shape[1]
    blk = 2048

    def body(g_ref, h_ref, d2_ref, bpre_ref, w_ref, wp_ref, t_ref, hp_ref):
        parts = []
        d2 = d2_ref[...]
        for p in range(npin):
            parts.append(g_ref[0, p] + g_ref[1, p] + d2 * h_ref[p]
                         + bpre_ref[:, p * HW:(p + 1) * HW])
        a = jnp.maximum(jnp.concatenate(parts, axis=-1), 0.0)
        r = jnp.dot(a, w_ref[...], preferred_element_type=jnp.float32)
        rp = jnp.dot(a, wp_ref[...], preferred_element_type=jnp.float32)
        t_ref[0] = r.astype(jnp.bfloat16)
        hp_ref[0] = rp

    return pl.pallas_call(
        body,
        grid=(NP // blk,),
        in_specs=[
            pl.BlockSpec((NC, npin, blk, HW), lambda i: (0, 0, i, 0)),
            pl.BlockSpec((npin, blk, HW), lambda i: (0, i, 0)),
            pl.BlockSpec((blk, 1), lambda i: (i, 0)),
            pl.BlockSpec((1, npin * HW), lambda i: (0, 0)),
            pl.BlockSpec((npin * HW, HW), lambda i: (0, 0)),
            pl.BlockSpec((npin * HW, HW), lambda i: (0, 0)),
        ],
        out_specs=[
            pl.BlockSpec((1, blk, HW), lambda i: (0, i, 0)),
            pl.BlockSpec((1, blk, HW), lambda i: (0, i, 0)),
        ],
        out_shape=[
            jax.ShapeDtypeStruct((1, NP, HW), jnp.bfloat16),
            jax.ShapeDtypeStruct((1, NP, HW), jnp.float32),
        ],
    )(gp, hperm, dinv2, b_pre, w, wp)


def _tc_comb2(gp, hperm, dinv2, b_pre, w, b_post):
    dout = w.shape[1]
    blk = 2048

    def body(g_ref, h_ref, d2_ref, bpre_ref, w_ref, bpost_ref, o_ref):
        t = (g_ref[0, 0] + g_ref[1, 0] + d2_ref[...] * h_ref[0]
             + bpre_ref[...])
        a = jnp.maximum(t, 0.0)
        o_ref[...] = (jnp.dot(a, w_ref[...],
                              preferred_element_type=jnp.float32)
                      + bpost_ref[...])

    return pl.pallas_call(
        body,
        grid=(NP // blk,),
        in_specs=[
            pl.BlockSpec((NC, 1, blk, HW), lambda i: (0, 0, i, 0)),
            pl.BlockSpec((1, blk, HW), lambda i: (0, i, 0)),
            pl.BlockSpec((blk, 1), lambda i: (i, 0)),
            pl.BlockSpec((1, HW), lambda i: (0, 0)),
            pl.BlockSpec((HW, dout), lambda i: (0, 0)),
            pl.BlockSpec((1, dout), lambda i: (0, 0)),
        ],
        out_specs=pl.BlockSpec((blk, dout), lambda i: (i, 0)),
        out_shape=jax.ShapeDtypeStruct((NP, dout), jnp.float32),
    )(gp, hperm, dinv2, b_pre, w, b_post)


def kernel(x, edge_index, edge_weight, W1, b1, W2, b2, Wl, bl):
    u32 = np.empty((32,), np.int32)
    u32[:16] = np.arange(16) * 2
    u32[16:] = np.arange(16) * 2 + 1
    u64 = np.concatenate([u32, u32 + 32])
    u128 = np.concatenate([u64, u64 + 64])

    pad = EP - E
    izero = jnp.zeros((pad,), edge_index.dtype)
    src2 = jnp.concatenate([edge_index[0], izero]).reshape(EROWS, CH)
    dst2 = jnp.concatenate([edge_index[1], izero]).reshape(EROWS, CH)
    w2 = jnp.concatenate(
        [edge_weight, jnp.zeros((pad,), edge_weight.dtype)]).reshape(EROWS, CH)

    wn2, d2 = _sc_pre(src2, dst2, w2)
    dinv2 = d2.reshape(NP, 1)

    xp = jnp.concatenate(
        [x, jnp.zeros((NP - N, x.shape[1]), x.dtype)], axis=0)
    h1t, h1p = _tc_mm_split(xp, W1, W1[:, u128])
    g1 = _sc_agg(h1t, src2, dst2, wn2, 2)
    W2u = W2[u128, :]
    h2t, h2p = _tc_comb1(g1, h1p, dinv2, b1[u128].reshape(1, -1),
                         W2u, W2u[:, u64])
    g2 = _sc_agg(h2t, src2, dst2, wn2, 1)
    out = _tc_comb2(g2, h2p, dinv2, b2[u64].reshape(1, -1),
                    Wl[u64, :], bl.reshape(1, -1))
    return out[:N]
